# Initial kernel scaffold; baseline (speedup 1.0000x reference)
import jax
import jax.numpy as jnp
from jax.experimental import pallas as pl


def kernel(token_embedding, prompts, token_prefix, ctx_embedding):
    raise NotImplementedError("write your pallas kernel here")


# speedup vs baseline: 1.0438x; 1.0438x over previous
import functools

import jax
import jax.numpy as jnp
from jax import lax
from jax.experimental import pallas as pl
from jax.experimental.pallas import tpu as pltpu
from jax.experimental.pallas import tpu_sc as plsc

_VOCAB = 49408
_D = 512
_NC = 1000
_CTX_LEN = 77
_N_CTX = 16
_SUF = _CTX_LEN - (_N_CTX + 1)
_SUF_PAD = 64
_PREFIX_ROWS = _CTX_LEN - _SUF

_NW = 32
_CPW = 32
_NC_PAD = _NW * _CPW


def _emb_body(table, prompts_p, prefix, ctx, out, eos, block, idx_v, eos_v, sem):
    wid = lax.axis_index("s") * 2 + lax.axis_index("c")
    base = wid * _CPW
    n_cls = jnp.minimum(_CPW, _NC - base)

    pltpu.sync_copy(ctx, block.at[pl.ds(1, _N_CTX)])

    def cls_body(i, carry):
        c = base + i
        pltpu.sync_copy(prompts_p.at[c], idx_v)
        gather = pltpu.async_copy(
            table.at[idx_v.at[pl.ds(0, _SUF)]],
            block.at[pl.ds(_PREFIX_ROWS, _SUF)],
            sem,
        )
        pltpu.sync_copy(prefix.at[c], block.at[pl.ds(0, 1)])

        best_val = jnp.int32(-2)
        best_pos = jnp.int32(0)
        for j in range(_SUF_PAD // 16):
            vj = idx_v[pl.ds(16 * j, 16)]
            mj = jnp.max(vj)
            lane_pos = lax.iota(jnp.int32, 16) + jnp.int32(16 * j)
            pj = jnp.min(jnp.where(vj == mj, lane_pos, jnp.int32(1 << 20)))
            upd = mj > best_val
            best_pos = jnp.where(upd, pj, best_pos)
            best_val = jnp.where(upd, mj, best_val)
        lane0 = lax.iota(jnp.int32, 16) == 0
        plsc.store_scatter(
            eos_v,
            [jnp.full((16,), i, jnp.int32)],
            jnp.full((16,), best_pos + jnp.int32(_PREFIX_ROWS), jnp.int32),
            mask=lane0,
        )

        gather.wait()
        pltpu.sync_copy(block, out.at[c])
        return carry

    lax.fori_loop(0, n_cls, cls_body, 0)
    pltpu.sync_copy(eos_v, eos.at[pl.ds(base, _CPW)])


@functools.partial(jax.jit, static_argnames=())
def _emb_call(table, prompts_p, prefix, ctx):
    mesh = plsc.VectorSubcoreMesh(core_axis_name="c", subcore_axis_name="s")
    return pl.kernel(
        _emb_body,
        out_type=[
            jax.ShapeDtypeStruct((_NC, _CTX_LEN, _D), jnp.float32),
            jax.ShapeDtypeStruct((_NC_PAD,), jnp.int32),
        ],
        mesh=mesh,
        scratch_types=[
            pltpu.VMEM((_CTX_LEN, _D), jnp.float32),
            pltpu.VMEM((_SUF_PAD,), jnp.int32),
            pltpu.VMEM((_CPW,), jnp.int32),
            pltpu.SemaphoreType.DMA,
        ],
        compiler_params=pltpu.CompilerParams(use_tc_tiling_on_sc=False,
                                             needs_layout_passes=False),
    )(table, prompts_p, prefix, ctx)


def kernel(token_embedding, prompts, token_prefix, ctx_embedding):
    prompts_i = prompts.astype(jnp.int32)
    prompts_p = jnp.pad(prompts_i, ((0, 0), (0, _SUF_PAD - _SUF)),
                        constant_values=-1)
    emb, eos = _emb_call(token_embedding, prompts_p, token_prefix,
                         ctx_embedding)
    return emb, eos[:_NC]


# speedup vs baseline: 1.0803x; 1.0350x over previous
import functools

import jax
import jax.numpy as jnp
from jax import lax
from jax.experimental import pallas as pl
from jax.experimental.pallas import tpu as pltpu
from jax.experimental.pallas import tpu_sc as plsc

_VOCAB = 49408
_D = 512
_NC = 1000
_CTX_LEN = 77
_N_CTX = 16
_SUF = _CTX_LEN - (_N_CTX + 1)
_SUF_PAD = 64
_PREFIX_ROWS = _CTX_LEN - _SUF

_NW = 32
_CPW = 32
_NC_PAD = _NW * _CPW


def _emb_body(table, prompts_p, prefix, ctx, out, eos,
              block0, block1, idx0, idx1, eos_v,
              sem_i0, sem_i1, sem_g0, sem_g1, sem_p0, sem_p1,
              sem_o0, sem_o1):
    wid = lax.axis_index("s") * 2 + lax.axis_index("c")
    base = wid * _CPW
    blocks = (block0, block1)
    idxs = (idx0, idx1)
    sem_i = (sem_i0, sem_i1)
    sem_g = (sem_g0, sem_g1)
    sem_p = (sem_p0, sem_p1)
    sem_o = (sem_o0, sem_o1)

    def cls(i):
        return jnp.minimum(base + i, jnp.int32(_NC - 1))

    pltpu.sync_copy(ctx, block0.at[pl.ds(1, _N_CTX)])
    pltpu.sync_copy(ctx, block1.at[pl.ds(1, _N_CTX)])

    pltpu.async_copy(prompts_p.at[cls(0)], idx0, sem_i0)

    def one_class(i, b):
        c = cls(i)
        @pl.when(i >= 2)
        def _():
            pltpu.make_async_copy(blocks[b], out.at[c], sem_o[b]).wait()
        pltpu.async_copy(prefix.at[c], blocks[b].at[pl.ds(0, 1)], sem_p[b])
        pltpu.make_async_copy(prompts_p.at[c], idxs[b], sem_i[b]).wait()
        pltpu.async_copy(
            table.at[idxs[b].at[pl.ds(0, _SUF)]],
            blocks[b].at[pl.ds(_PREFIX_ROWS, _SUF)],
            sem_g[b],
        )
        pltpu.async_copy(prompts_p.at[cls(i + 1)], idxs[1 - b], sem_i[1 - b])

        best_val = jnp.int32(-2)
        best_pos = jnp.int32(0)
        for j in range(_SUF_PAD // 16):
            vj = idxs[b][pl.ds(16 * j, 16)]
            mj = jnp.max(vj)
            lane_pos = lax.iota(jnp.int32, 16) + jnp.int32(16 * j)
            pj = jnp.min(jnp.where(vj == mj, lane_pos, jnp.int32(1 << 20)))
            upd = mj > best_val
            best_pos = jnp.where(upd, pj, best_pos)
            best_val = jnp.where(upd, mj, best_val)
        lane0 = lax.iota(jnp.int32, 16) == 0
        plsc.store_scatter(
            eos_v,
            [jnp.full((16,), i, jnp.int32)],
            jnp.full((16,), best_pos + jnp.int32(_PREFIX_ROWS), jnp.int32),
            mask=lane0,
        )

        pltpu.make_async_copy(
            table.at[idxs[b].at[pl.ds(0, _SUF)]],
            blocks[b].at[pl.ds(_PREFIX_ROWS, _SUF)],
            sem_g[b],
        ).wait()
        pltpu.make_async_copy(prefix.at[c], blocks[b].at[pl.ds(0, 1)],
                              sem_p[b]).wait()
        pltpu.async_copy(blocks[b], out.at[c], sem_o[b])

    def pair(k, carry):
        one_class(2 * k, 0)
        one_class(2 * k + 1, 1)
        return carry

    lax.fori_loop(0, _CPW // 2, pair, 0)

    last = cls(_CPW - 1)
    pltpu.make_async_copy(block0, out.at[last], sem_o0).wait()
    pltpu.make_async_copy(block1, out.at[last], sem_o1).wait()
    pltpu.make_async_copy(prompts_p.at[last], idx0, sem_i0).wait()
    pltpu.sync_copy(eos_v, eos.at[pl.ds(base, _CPW)])


@functools.partial(jax.jit, static_argnames=())
def _emb_call(table, prompts_p, prefix, ctx):
    mesh = plsc.VectorSubcoreMesh(core_axis_name="c", subcore_axis_name="s")
    return pl.kernel(
        _emb_body,
        out_type=[
            jax.ShapeDtypeStruct((_NC, _CTX_LEN, _D), jnp.float32),
            jax.ShapeDtypeStruct((_NC_PAD,), jnp.int32),
        ],
        mesh=mesh,
        scratch_types=[
            pltpu.VMEM((_CTX_LEN, _D), jnp.float32),
            pltpu.VMEM((_CTX_LEN, _D), jnp.float32),
            pltpu.VMEM((_SUF_PAD,), jnp.int32),
            pltpu.VMEM((_SUF_PAD,), jnp.int32),
            pltpu.VMEM((_CPW,), jnp.int32),
            pltpu.SemaphoreType.DMA,
            pltpu.SemaphoreType.DMA,
            pltpu.SemaphoreType.DMA,
            pltpu.SemaphoreType.DMA,
            pltpu.SemaphoreType.DMA,
            pltpu.SemaphoreType.DMA,
            pltpu.SemaphoreType.DMA,
            pltpu.SemaphoreType.DMA,
        ],
        compiler_params=pltpu.CompilerParams(use_tc_tiling_on_sc=False,
                                             needs_layout_passes=False),
    )(table, prompts_p, prefix, ctx)


def kernel(token_embedding, prompts, token_prefix, ctx_embedding):
    prompts_i = prompts.astype(jnp.int32)
    prompts_p = jnp.pad(prompts_i, ((0, 0), (0, _SUF_PAD - _SUF)),
                        constant_values=-1)
    emb, eos = _emb_call(token_embedding, prompts_p, token_prefix,
                         ctx_embedding)
    return emb, eos[:_NC]


# speedup vs baseline: 7.3842x; 6.8353x over previous
import functools

import jax
import jax.numpy as jnp
from jax import lax
from jax.experimental import pallas as pl
from jax.experimental.pallas import tpu as pltpu
from jax.experimental.pallas import tpu_sc as plsc

_VOCAB = 49408
_D = 512
_NC = 1000
_CTX_LEN = 77
_N_CTX = 16
_SUF = _CTX_LEN - (_N_CTX + 1)
_SUF_PAD = 64
_PREFIX_ROWS = _CTX_LEN - _SUF

_NW = 32
_CPW = 32
_NC_PAD = _NW * _CPW

_JD = _D // 128
_NU_SUF = _SUF * _JD
_NSPLIT = 3
_USPL = _NU_SUF // _NSPLIT
_ROW_TILES = _NC // 8
_UNITS_PER_T = _ROW_TILES * _JD * 8
_PFX_WORKERS = 25
_PFX_CHUNK = _UNITS_PER_T // _PFX_WORKERS


def _emb_body(tab_u, prompts_p, pfx_u, ctx_u, out_u, eos,
              gbuf0, gbuf1, gidx0, gidx1, sidx0, sidx1, sbase,
              idx0, idx1, tmpl, cidx, pfx_v, eos_v,
              sem_i0, sem_i1, sem_g0, sem_g1, sem_s0, sem_s1,
              sem_ctx, sem_pfx, sem_eos):
    wid = lax.axis_index("s") * 2 + lax.axis_index("c")
    base = wid * _CPW
    gbufs = (gbuf0, gbuf1)
    gidxs = (gidx0, gidx1)
    sidxs = (sidx0, sidx1)
    idxs = (idx0, idx1)
    sem_i = (sem_i0, sem_i1)
    sem_g = (sem_g0, sem_g1)
    sem_s = (sem_s0, sem_s1)

    lanes = lax.iota(jnp.int32, 16)

    def cls(i):
        return jnp.minimum(base + i, jnp.int32(_NC - 1))

    pfx_off = jnp.minimum(wid, jnp.int32(_PFX_WORKERS - 1)) * _PFX_CHUNK

    @pl.when(wid < _PFX_WORKERS)
    def _():
        pltpu.async_copy(pfx_u.at[pl.ds(pfx_off, _PFX_CHUNK)], pfx_v, sem_pfx)

    tct = wid >> 1
    half = wid & 1
    for m in range(2):
        r = lanes + 16 * m
        u = 32 * (tct >> 3) + 8 * (r >> 3) + (tct & 7)
        cidx[pl.ds(16 * m, 16)] = u
    pltpu.async_copy(ctx_u.at[cidx], tmpl, sem_ctx).wait()

    n_rep = jnp.int32(63 - half)
    rep0 = jnp.int32(63) * half
    t_base = jnp.int32(_UNITS_PER_T) * (tct + 1) + 32 * rep0

    def ctx_rep(r, carry):
        off = pl.multiple_of(t_base + 32 * r, 8)
        pltpu.async_copy(tmpl, out_u.at[pl.ds(off, 32)], sem_ctx)
        return carry

    lax.fori_loop(0, n_rep, ctx_rep, 0)

    for j in range(_NSPLIT):
        for m in range(_USPL // 16):
            k = 80 * j + 16 * m + lanes
            sbase[j, pl.ds(16 * m, 16)] = (
                jnp.int32(_UNITS_PER_T) * (_PREFIX_ROWS + (k >> 2))
                + 8 * (k & 3))

    @pl.when(wid < _PFX_WORKERS)
    def _():
        pltpu.make_async_copy(pfx_u.at[pl.ds(pfx_off, _PFX_CHUNK)], pfx_v,
                              sem_pfx).wait()
        pltpu.async_copy(pfx_v, out_u.at[pl.ds(pfx_off, _PFX_CHUNK)], sem_pfx)

    pltpu.async_copy(prompts_p.at[cls(0)], idx0, sem_i0)

    def one_class(i, b):
        c = cls(i)
        offc = 32 * (c >> 3) + (c & 7)
        @pl.when(i >= 2)
        def _():
            for j in range(_NSPLIT):
                pltpu.make_async_copy(gbufs[b].at[j], out_u.at[sidxs[b].at[j]],
                                      sem_s[b]).wait()
        pltpu.make_async_copy(prompts_p.at[c], idxs[b], sem_i[b]).wait()
        for j in range(_NSPLIT):
            for m in range(_USPL // 16):
                k = 80 * j + 16 * m + lanes
                x = plsc.load_gather(idxs[b], [k >> 2])
                gidxs[b][j, pl.ds(16 * m, 16)] = (
                    32 * (x >> 3) + 8 * (k & 3) + (x & 7))
                sidxs[b][j, pl.ds(16 * m, 16)] = (
                    sbase[j, pl.ds(16 * m, 16)] + offc)
            pltpu.async_copy(tab_u.at[gidxs[b].at[j]], gbufs[b].at[j],
                             sem_g[b])
        pltpu.async_copy(prompts_p.at[cls(i + 1)], idxs[1 - b], sem_i[1 - b])

        best_val = jnp.int32(-2)
        best_pos = jnp.int32(0)
        for m in range(_SUF_PAD // 16):
            vj = idxs[b][pl.ds(16 * m, 16)]
            mj = jnp.max(vj)
            pj = jnp.min(jnp.where(vj == mj, lanes + jnp.int32(16 * m),
                                   jnp.int32(1 << 20)))
            upd = mj > best_val
            best_pos = jnp.where(upd, pj, best_pos)
            best_val = jnp.where(upd, mj, best_val)
        plsc.store_scatter(
            eos_v,
            [jnp.full((16,), i, jnp.int32)],
            jnp.full((16,), best_pos + jnp.int32(_PREFIX_ROWS), jnp.int32),
            mask=lanes == 0,
        )

        for j in range(_NSPLIT):
            pltpu.make_async_copy(tab_u.at[gidxs[b].at[j]], gbufs[b].at[j],
                                  sem_g[b]).wait()
        for j in range(_NSPLIT):
            pltpu.async_copy(gbufs[b].at[j], out_u.at[sidxs[b].at[j]],
                             sem_s[b])

    def pair(k, carry):
        one_class(2 * k, 0)
        one_class(2 * k + 1, 1)
        return carry

    lax.fori_loop(0, _CPW // 2, pair, 0)

    for b in range(2):
        for j in range(_NSPLIT):
            pltpu.make_async_copy(gbufs[b].at[j], out_u.at[sidxs[b].at[j]],
                                  sem_s[b]).wait()
    pltpu.make_async_copy(prompts_p.at[cls(0)], idx0, sem_i0).wait()

    def ctx_drain(r, carry):
        pltpu.make_async_copy(tmpl, out_u.at[pl.ds(0, 32)], sem_ctx).wait()
        return carry

    lax.fori_loop(0, n_rep, ctx_drain, 0)

    @pl.when(wid < _PFX_WORKERS)
    def _():
        pltpu.make_async_copy(pfx_v, out_u.at[pl.ds(pfx_off, _PFX_CHUNK)],
                              sem_pfx).wait()

    pltpu.async_copy(eos_v, eos.at[pl.ds(base, _CPW)], sem_eos).wait()


@functools.partial(jax.jit, static_argnames=())
def _emb_call(tab_u, prompts_p, pfx_u, ctx_u):
    mesh = plsc.VectorSubcoreMesh(core_axis_name="c", subcore_axis_name="s")
    return pl.kernel(
        _emb_body,
        out_type=[
            jax.ShapeDtypeStruct((_CTX_LEN * _UNITS_PER_T, 128), jnp.float32),
            jax.ShapeDtypeStruct((_NC_PAD,), jnp.int32),
        ],
        mesh=mesh,
        scratch_types=[
            pltpu.VMEM((_NSPLIT, _USPL, 128), jnp.float32),
            pltpu.VMEM((_NSPLIT, _USPL, 128), jnp.float32),
            pltpu.VMEM((_NSPLIT, _USPL), jnp.int32),
            pltpu.VMEM((_NSPLIT, _USPL), jnp.int32),
            pltpu.VMEM((_NSPLIT, _USPL), jnp.int32),
            pltpu.VMEM((_NSPLIT, _USPL), jnp.int32),
            pltpu.VMEM((_NSPLIT, _USPL), jnp.int32),
            pltpu.VMEM((_SUF_PAD,), jnp.int32),
            pltpu.VMEM((_SUF_PAD,), jnp.int32),
            pltpu.VMEM((32, 128), jnp.float32),
            pltpu.VMEM((32,), jnp.int32),
            pltpu.VMEM((_PFX_CHUNK, 128), jnp.float32),
            pltpu.VMEM((_CPW,), jnp.int32),
            pltpu.SemaphoreType.DMA,
            pltpu.SemaphoreType.DMA,
            pltpu.SemaphoreType.DMA,
            pltpu.SemaphoreType.DMA,
            pltpu.SemaphoreType.DMA,
            pltpu.SemaphoreType.DMA,
            pltpu.SemaphoreType.DMA,
            pltpu.SemaphoreType.DMA,
            pltpu.SemaphoreType.DMA,
        ],
        compiler_params=pltpu.CompilerParams(use_tc_tiling_on_sc=False,
                                             needs_layout_passes=False),
    )(tab_u, prompts_p, pfx_u, ctx_u)


def kernel(token_embedding, prompts, token_prefix, ctx_embedding):
    tab_u = (token_embedding.reshape(_VOCAB // 8, 8, _JD, 128)
             .transpose(0, 2, 1, 3).reshape(_VOCAB * _JD, 128))
    pfx_u = (token_prefix.reshape(_NC // 8, 8, _JD, 128)
             .transpose(0, 2, 1, 3).reshape(_NC * _JD, 128))
    ctx_u = (ctx_embedding.reshape(_N_CTX // 8, 8, _JD, 128)
             .transpose(0, 2, 1, 3).reshape(_N_CTX * _JD, 128))
    prompts_i = prompts.astype(jnp.int32)
    prompts_p = jnp.pad(prompts_i, ((0, 0), (0, _SUF_PAD - _SUF)),
                        constant_values=-1)
    out_u, eos = _emb_call(tab_u, prompts_p, pfx_u, ctx_u)
    emb = (out_u.reshape(_CTX_LEN, _ROW_TILES, _JD, 8, 128)
           .transpose(1, 3, 0, 2, 4).reshape(_NC, _CTX_LEN, _D))
    return emb, eos[:_NC]
